# baseline (speedup 1.0000x reference)
import functools

import jax
import jax.numpy as jnp
from jax import lax
from jax.experimental import pallas as pl
from jax.experimental.pallas import tpu as pltpu
from jax.experimental.pallas import tpu_sc as plsc

N = 10000
E1 = 160000
EA = 320000
S2 = 5000
I2 = 50000
D = 128
L = 6
D0 = 57

NP = N + 16
K = 64
W = 128
AXF = 30720
EA_PAD = 327680
I1_PAD = 327680
I2_PAD = 65536
ADJ_CHUNKS = EA_PAD // (16 * K)
I1_CHUNKS = I1_PAD // (16 * K)
I2_CHUNKS = I2_PAD // (16 * K)
TROWS = 632
ACC_ROWS = 16 * TROWS


def _embed_body(x_ref, w_ref, b_ref, o_ref):
    o_ref[...] = jnp.dot(x_ref[...], w_ref[...],
                         preferred_element_type=jnp.float32,
                 precision=jax.lax.Precision.HIGHEST) + b_ref[...]


def _embed(x, w, b):
    return pl.pallas_call(
        _embed_body,
        grid=(10,),
        in_specs=[pl.BlockSpec((1000, D0), lambda i: (i, 0)),
                  pl.BlockSpec((D0, D), lambda i: (0, 0)),
                  pl.BlockSpec((D,), lambda i: (0,))],
        out_specs=pl.BlockSpec((1000, D), lambda i: (i, 0)),
        out_shape=jax.ShapeDtypeStruct((N, D), jnp.float32),
    )(x, w, b)


def _proj_body(h_ref, w_ref, b_ref, o_ref):
    o_ref[...] = jnp.dot(h_ref[...], w_ref[...],
                         preferred_element_type=jnp.float32,
                 precision=jax.lax.Precision.HIGHEST) + b_ref[...]


def _proj(h, wc, bc):
    return pl.pallas_call(
        _proj_body,
        grid=(10,),
        in_specs=[pl.BlockSpec((1000, D), lambda i: (i, 0)),
                  pl.BlockSpec((D, 6 * D), lambda i: (0, 0)),
                  pl.BlockSpec((6 * D,), lambda i: (0,))],
        out_specs=pl.BlockSpec((1000, 6 * D), lambda i: (i, 0)),
        out_shape=jax.ShapeDtypeStruct((N, 6 * D), jnp.float32),
    )(h, wc, bc)


def _q_body(e_ref, w_ref, o_ref):
    e = e_ref[...]
    w = w_ref[...]
    acc = e[:, 0:1] * w[0:1, :]
    for j in range(1, w.shape[0]):
        acc = acc + e[:, j:j + 1] * w[j:j + 1, :]
    o_ref[...] = acc


def _q(attr, wq, rows, block):
    nb = rows // block
    da = attr.shape[1]
    return pl.pallas_call(
        _q_body,
        grid=(nb,),
        in_specs=[pl.BlockSpec((block, da), lambda i: (i, 0)),
                  pl.BlockSpec((da, D), lambda i: (0, 0))],
        out_specs=pl.BlockSpec((block, D), lambda i: (i, 0)),
        out_shape=jax.ShapeDtypeStruct((rows, D), jnp.float32),
    )(attr, wq)


def _upd_body(h_ref, o1_ref, o2_ref, i1a_ref, i1b_ref, i2_ref, x_ref, ax_ref,
              wh_ref, bh_ref, ho_ref, xo_ref):
    h = h_ref[...]
    wh = wh_ref[...]
    hp = jax.lax.Precision.HIGHEST
    u = jnp.dot(h, wh[0:D], preferred_element_type=jnp.float32, precision=hp)
    u = u + jnp.dot(o1_ref[...], wh[D:2 * D],
                    preferred_element_type=jnp.float32, precision=hp)
    u = u + jnp.dot(o2_ref[...], wh[2 * D:3 * D],
                    preferred_element_type=jnp.float32, precision=hp)
    u = u + jnp.dot(i1a_ref[...] + i1b_ref[...], wh[3 * D:4 * D],
                    preferred_element_type=jnp.float32, precision=hp)
    u = u + jnp.dot(i2_ref[...], wh[4 * D:5 * D],
                    preferred_element_type=jnp.float32, precision=hp)
    u = jnp.maximum(u + bh_ref[...], 0.0)
    ho_ref[...] = h + u
    xo_ref[...] = x_ref[...] + 0.001 * ax_ref[...]


def _update(h, o1, o2, i1a, i1b, i2, x, axn, wh, bh):
    return pl.pallas_call(
        _upd_body,
        grid=(10,),
        in_specs=[pl.BlockSpec((1000, D), lambda i: (i, 0)),
                  pl.BlockSpec((1000, W), lambda i: (i, 0)),
                  pl.BlockSpec((1000, W), lambda i: (i, 0)),
                  pl.BlockSpec((1000, W), lambda i: (i, 0)),
                  pl.BlockSpec((1000, W), lambda i: (i, 0)),
                  pl.BlockSpec((1000, W), lambda i: (i, 0)),
                  pl.BlockSpec((1000, 3), lambda i: (i, 0)),
                  pl.BlockSpec((1000, 3), lambda i: (i, 0)),
                  pl.BlockSpec((5 * D, D), lambda i: (0, 0)),
                  pl.BlockSpec((D,), lambda i: (0,))],
        out_specs=[pl.BlockSpec((1000, D), lambda i: (i, 0)),
                   pl.BlockSpec((1000, 3), lambda i: (i, 0))],
        out_shape=[jax.ShapeDtypeStruct((N, D), jnp.float32),
                   jax.ShapeDtypeStruct((N, 3), jnp.float32)],
    )(h, o1, o2, i1a, i1b, i2, x, axn, wh, bh)


_SC_MESH = plsc.VectorSubcoreMesh(core_axis_name="c", subcore_axis_name="s",
                                  num_cores=2, num_subcores=16)
_SC_PARAMS = pltpu.CompilerParams(needs_layout_passes=False)

EPT = EA_PAD // 16
KG = 512
NGC = EPT // KG
I1A_CH = 128
I1B_CH = 192
I1B_BASE = 16 * I1A_CH * K


def _geom_body(s1, t1, s2, t2, Xp, g1, g2, Xv, sbig, tbig, d2g, dxg):
    cid = lax.axis_index("c")
    sid = lax.axis_index("s")
    pltpu.sync_copy(Xp, Xv)

    def pass_(s_hbm, t_hbm, g_hbm):
        base = sid * EPT
        pltpu.sync_copy(s_hbm.at[pl.ds(base, EPT)], sbig)
        pltpu.sync_copy(t_hbm.at[pl.ds(base, EPT)], tbig)

        def chunk(k, carry):
            off = k * KG
            for i in range(KG // 16):
                sv = sbig[pl.ds(off + i * 16, 16)]
                tv = tbig[pl.ds(off + i * 16, 16)]
                d2 = jnp.zeros((16,), jnp.float32)
                for c in range(3):
                    dx = (plsc.load_gather(Xv, [tv + c * NP])
                          - plsc.load_gather(Xv, [sv + c * NP]))
                    dxg[pl.ds(c * KG + i * 16, 16)] = dx
                    d2 = d2 + dx * dx
                d2g[pl.ds(i * 16, 16)] = d2
            pltpu.sync_copy(d2g, g_hbm.at[pl.ds(base + off, KG)])
            pltpu.sync_copy(
                dxg, g_hbm.at[pl.ds(EA_PAD + 3 * (base + off), 3 * KG)])
            return carry
        lax.fori_loop(0, NGC, chunk, 0)

    @pl.when(cid == 0)
    def _():
        pass_(s1, t1, g1)

    @pl.when(cid == 1)
    def _():
        pass_(s2, t2, g2)


_sc_geom = pl.kernel(
    _geom_body,
    out_type=[jax.ShapeDtypeStruct((4 * EA_PAD,), jnp.float32)
              for _ in range(2)],
    mesh=_SC_MESH,
    scratch_types=[
        pltpu.VMEM((3 * NP,), jnp.float32),
        pltpu.VMEM((EPT,), jnp.int32),
        pltpu.VMEM((EPT,), jnp.int32),
        pltpu.VMEM((KG,), jnp.float32),
        pltpu.VMEM((3 * KG,), jnp.float32),
    ],
    compiler_params=_SC_PARAMS,
)


def _msg_body(st1, t1_2d, st2, t2_2d, n1_2d, cn1, n2_2d, cn2,
              AB1, AB2, PQ1, PQ2, g1, g2, wpack,
              o1, o2, i1hA, i1hB, i2h, u1, u2,
              sbig, tbig, d2big, ubig, ab0, ab1, wv, acc,
              sga0, ssc0, sga1, ssc1):
    cid = lax.axis_index("c")
    sid = lax.axis_index("s")
    zero16 = jnp.zeros((16,), jnp.float32)
    lanes = lax.iota(jnp.int32, 16)
    lane0 = lanes == 0
    SC = 8
    SE = SC * K

    pltpu.sync_copy(wpack, wv)

    AB = (ab0, ab1)
    SGA = (sga0, sga1)
    SSC = (ssc0, ssc1)

    def zero_ar0():
        def row(e, carry):
            for j in range(8):
                ab0[e, pl.ds(j * 16, 16)] = zero16
            return carry
        lax.fori_loop(0, K, row, 0)

    def zero_acc():
        zsrc = ab0.at[pl.ds(0, K)]
        base = sid * TROWS
        for j in range(TROWS // K):
            pltpu.sync_copy(zsrc, acc.at[pl.ds(base + j * K, K)])
        rem = TROWS % K
        pltpu.sync_copy(ab0.at[pl.ds(0, rem)],
                        acc.at[pl.ds(base + TROWS - rem, rem)])

    def flush(out_ref):
        base = sid * TROWS
        pltpu.sync_copy(acc.at[pl.ds(base, TROWS)],
                        out_ref.at[pl.ds(base, TROWS)])

    def super_pass(nsuper, ebase, rbase, st_hbm, t2d_hbm, AB_hbm, compute,
                   g_hbm, u_hbm):
        dsrc = AB_hbm.at[pl.ds(0, 2 * K)]
        dsrc_k = AB_hbm.at[pl.ds(0, K)]

        def wait_g(p):
            pltpu.make_async_copy(dsrc, AB[p], SGA[p]).wait()

        def wait_s(p):
            pltpu.make_async_copy(dsrc_k, AB[p].at[pl.ds(K, K)],
                                  SSC[p]).wait()

        def fetch(cc, p):
            pltpu.async_copy(AB_hbm.at[sbig.at[pl.ds(cc * 2 * K, 2 * K)]],
                             AB[p], SGA[p])

        def super(j, carry):
            eb = ebase + j * SE
            pltpu.sync_copy(st_hbm.at[pl.ds(2 * eb, 2 * SE)], sbig)
            pltpu.sync_copy(
                t2d_hbm.at[pl.ds(pl.multiple_of(rbase + j * SC, 8), SC)],
                tbig)
            if g_hbm is not None:
                pltpu.sync_copy(g_hbm.at[pl.ds(eb, SE)], d2big)

            @pl.when(j > 0)
            def _():
                wait_s(0)
            fetch(0, 0)
            for cc in range(SC):
                p = cc % 2
                wait_g(p)
                if cc < SC - 1:
                    q = 1 - p
                    if cc == 0:
                        @pl.when(j > 0)
                        def _():
                            wait_s(1)
                    else:
                        wait_s(q)
                    fetch(cc + 1, q)
                compute(cc, AB[p])
                pltpu.async_copy(AB[p].at[pl.ds(K, K)], acc.at[tbig.at[cc]],
                                 SSC[p], add=True)
            if u_hbm is not None:
                for i in range(SE // 16):
                    uv = ubig[pl.ds(i * 16, 16)]
                    e2v = jnp.exp(-2.0 * jnp.abs(uv))
                    th = (1.0 - e2v) / (1.0 + e2v)
                    ubig[pl.ds(i * 16, 16)] = jnp.where(uv < 0.0, -th, th)
                pltpu.sync_copy(ubig, u_hbm.at[pl.ds(eb, SE)])
            return carry
        lax.fori_loop(0, nsuper, super, 0)
        wait_s(0)
        wait_s(1)

    def adj_pass(st_hbm, t2d_hbm, AB_hbm, g_hbm, u_hbm, wd2_row, wx_row):
        wd2 = [wv[wd2_row, pl.ds(j * 16, 16)] for j in range(8)]
        wx = [wv[wx_row, pl.ds(j * 16, 16)] for j in range(8)]

        def compute(cc, ab):
            def edge2(e2, ecarry):
                for half in range(4):
                    e = 4 * e2 + half
                    ev = jnp.full((16,), e + cc * K, jnp.int32)
                    d2v = plsc.load_gather(d2big, [ev])
                    u = zero16
                    for j in range(8):
                        m = jnp.maximum(
                            ab[e, pl.ds(j * 16, 16)]
                            + ab[K + e, pl.ds(j * 16, 16)]
                            + d2v * wd2[j], 0.0)
                        ab[K + e, pl.ds(j * 16, 16)] = m
                        u = u + m * wx[j]
                    plsc.store_scatter(ubig, [ev],
                                       jnp.full((16,), jnp.sum(u),
                                                jnp.float32),
                                       mask=lane0)
                return ecarry
            lax.fori_loop(0, K // 4, edge2, 0)

        super_pass(ADJ_CHUNKS // SC, sid * EPT, sid * (EPT // K),
                   st_hbm, t2d_hbm, AB_hbm, compute, g_hbm, u_hbm)

    def inc_pass(cn_hbm, n2d_hbm, PQ_hbm, nch, ebase):
        def compute(cc, ab):
            def edge(e, ecarry):
                for j in range(8):
                    ab[K + e, pl.ds(j * 16, 16)] = jnp.maximum(
                        ab[e, pl.ds(j * 16, 16)]
                        + ab[K + e, pl.ds(j * 16, 16)], 0.0)
                return ecarry
            lax.fori_loop(0, K, edge, 0)

        super_pass(nch // SC, ebase, ebase // K,
                   cn_hbm, n2d_hbm, PQ_hbm, compute, None, None)

    @pl.when(cid == 0)
    def _():
        zero_ar0()
        zero_acc()
        plsc.subcore_barrier()
        adj_pass(st1, t1_2d, AB1, g1, u1, 0, 1)
        plsc.subcore_barrier()
        flush(o1)
        plsc.subcore_barrier()
        zero_ar0()
        zero_acc()
        plsc.subcore_barrier()
        inc_pass(cn1, n1_2d, PQ1, I1A_CH, sid * (I1A_CH * K))
        plsc.subcore_barrier()
        flush(i1hA)
        plsc.subcore_barrier()
        zero_ar0()
        zero_acc()
        plsc.subcore_barrier()
        inc_pass(cn2, n2_2d, PQ2, I2_CHUNKS,
                 sid * (I2_CHUNKS * K))
        plsc.subcore_barrier()
        flush(i2h)

    @pl.when(cid == 1)
    def _():
        zero_ar0()
        zero_acc()
        plsc.subcore_barrier()
        adj_pass(st2, t2_2d, AB2, g2, u2, 2, 3)
        plsc.subcore_barrier()
        flush(o2)
        plsc.subcore_barrier()
        zero_ar0()
        zero_acc()
        plsc.subcore_barrier()
        inc_pass(cn1, n1_2d, PQ1, I1B_CH,
                 I1B_BASE + sid * (I1B_CH * K))
        plsc.subcore_barrier()
        flush(i1hB)


_sc_msg = pl.kernel(
    _msg_body,
    out_type=[jax.ShapeDtypeStruct((ACC_ROWS, W), jnp.float32)
              for _ in range(5)]
    + [jax.ShapeDtypeStruct((EA_PAD,), jnp.float32) for _ in range(2)],
    mesh=_SC_MESH,
    scratch_types=[
        pltpu.VMEM((1024,), jnp.int32),
        pltpu.VMEM((8, K), jnp.int32),
        pltpu.VMEM((512,), jnp.float32),
        pltpu.VMEM((512,), jnp.float32),
        pltpu.VMEM((2 * K, D), jnp.float32),
        pltpu.VMEM((2 * K, D), jnp.float32),
        pltpu.VMEM((4, D), jnp.float32),
        pltpu.VMEM_SHARED((ACC_ROWS, W), jnp.float32),
        pltpu.SemaphoreType.DMA,
        pltpu.SemaphoreType.DMA,
        pltpu.SemaphoreType.DMA,
        pltpu.SemaphoreType.DMA,
    ],
    compiler_params=_SC_PARAMS,
)


def _ax_body(t1, t2, u1, u2, g1, g2, ax1p, ax2p, axv, tbig, ubig, dxg):
    cid = lax.axis_index("c")
    sid = lax.axis_index("s")
    zero16 = jnp.zeros((16,), jnp.float32)
    lanes = lax.iota(jnp.int32, 16)
    axmask = lanes < 3
    plane = lanes * NP

    def pass_(t_hbm, u_hbm, g_hbm, axp):
        def zrow(i, carry):
            axv[pl.ds(i * 16, 16)] = zero16
            return carry
        lax.fori_loop(0, AXF // 16, zrow, 0)
        base = sid * EPT
        pltpu.sync_copy(t_hbm.at[pl.ds(base, EPT)], tbig)
        pltpu.sync_copy(u_hbm.at[pl.ds(base, EPT)], ubig)

        def chunk(k, carry):
            off = k * KG
            pltpu.sync_copy(
                g_hbm.at[pl.ds(EA_PAD + 3 * (base + off), 3 * KG)], dxg)

            def grp(i, ecarry):
                tv = tbig[pl.ds(off + i * 16, 16)]
                uv = ubig[pl.ds(off + i * 16, 16)]
                for c in range(3):
                    dv = dxg[pl.ds(c * KG + i * 16, 16)]
                    plsc.addupdate_scatter(axv, [tv + c * NP], uv * dv)
                return ecarry
            lax.fori_loop(0, KG // 16, grp, 0, unroll=2)
            return carry
        lax.fori_loop(0, NGC, chunk, 0)
        pltpu.sync_copy(axv, axp.at[sid])

    @pl.when(cid == 0)
    def _():
        pass_(t1, u1, g1, ax1p)

    @pl.when(cid == 1)
    def _():
        pass_(t2, u2, g2, ax2p)


_sc_ax = pl.kernel(
    _ax_body,
    out_type=[jax.ShapeDtypeStruct((16, AXF), jnp.float32)
              for _ in range(2)],
    mesh=_SC_MESH,
    scratch_types=[
        pltpu.VMEM((AXF,), jnp.float32),
        pltpu.VMEM((EPT,), jnp.int32),
        pltpu.VMEM((EPT,), jnp.float32),
        pltpu.VMEM((3 * KG,), jnp.float32),
    ],
    compiler_params=_SC_PARAMS,
)


def _pad_idx(a, n, fill):
    return jnp.concatenate(
        [a, jnp.full((n - a.shape[0],), fill, jnp.int32)])


def _pad_rows(a):
    return jnp.pad(a, ((0, 16), (0, 0)))


def kernel(x, pos, edge_attr, sse_attr, adj1_src, adj1_dst, adj2_src,
           adj2_dst, inc1_cell, inc1_node, inc2_cell, inc2_node,
           W0, b0, Wm1, bm1, Wx1, Wm2, bm2, Wx2, Wi1, bi1, Wi2, bi2, Wh, bh):
    H0 = _embed(x, W0, b0)
    X = pos

    s1 = _pad_idx(adj1_src, EA_PAD, N)
    t1 = _pad_idx(adj1_dst, EA_PAD, N)
    s2 = _pad_idx(adj2_src, EA_PAD, N)
    t2 = _pad_idx(adj2_dst, EA_PAD, N)
    c1 = _pad_idx(inc1_cell, I1_PAD, 0)
    n1 = _pad_idx(inc1_node, I1_PAD, N)
    c2 = _pad_idx(inc2_cell, I2_PAD, 0)
    n2 = _pad_idx(inc2_node, I2_PAD, N)
    t1_2d = t1.reshape(-1, K)
    t2_2d = t2.reshape(-1, K)
    n1_2d = n1.reshape(-1, K)
    n2_2d = n2.reshape(-1, K)
    def _comb(a, b, boff):
        return jnp.concatenate(
            [a.reshape(-1, K), b.reshape(-1, K) + boff], axis=1).reshape(-1)
    st1 = _comb(s1, t1, NP)
    st2 = _comb(s2, t2, NP)
    cn1 = _comb(c1, n1, E1)
    cn2 = _comb(c2, n2, S2)

    zD = jnp.zeros((D,), jnp.float32)
    for l in range(L):
        Wc = jnp.concatenate([
            Wm1[l, :D], Wm1[l, D:2 * D],
            Wm2[l, :D], Wm2[l, D:2 * D],
            Wi1[l, :D], Wi2[l, :D]], axis=1)
        bc = jnp.concatenate([zD, bm1[l], zD, bm2[l], bi1[l], bi2[l]])
        C = _proj(H0, Wc, bc)
        A1 = _pad_rows(C[:, 0:D])
        B1 = _pad_rows(C[:, D:2 * D])
        A2 = _pad_rows(C[:, 2 * D:3 * D])
        B2 = _pad_rows(C[:, 3 * D:4 * D])
        P1 = _pad_rows(C[:, 4 * D:5 * D])
        P2 = _pad_rows(C[:, 5 * D:6 * D])
        Q1 = _q(edge_attr, Wi1[l, D:], E1, 2000)
        Q2 = _q(sse_attr, Wi2[l, D:], S2, 5000)
        Xp = jnp.pad(X, ((0, 16), (0, 0))).T.reshape(-1)
        wpack = jnp.stack([Wm1[l, 2 * D], Wx1[l, :, 0],
                           Wm2[l, 2 * D], Wx2[l, :, 0]])

        g1, g2 = _sc_geom(s1, t1, s2, t2, Xp)
        AB1 = jnp.concatenate([A1, B1])
        AB2 = jnp.concatenate([A2, B2])
        PQ1 = jnp.concatenate([Q1, P1])
        PQ2 = jnp.concatenate([Q2, P2])
        o1, o2, i1hA, i1hB, i2h, u1, u2 = _sc_msg(
            st1, t1_2d, st2, t2_2d, n1_2d, cn1, n2_2d, cn2,
            AB1, AB2, PQ1, PQ2, g1, g2, wpack)
        ax1p, ax2p = _sc_ax(t1, t2, u1, u2, g1, g2)

        axf = ax1p.sum(axis=0) + ax2p.sum(axis=0)
        axn = axf[:3 * NP].reshape(3, NP)[:, :N].T

        H0, X = _update(H0, o1, o2, i1hA, i1hB, i2h, X, axn, Wh[l], bh[l])

    return H0, X


# speedup vs baseline: 1.0190x; 1.0190x over previous
import functools

import jax
import jax.numpy as jnp
from jax import lax
from jax.experimental import pallas as pl
from jax.experimental.pallas import tpu as pltpu
from jax.experimental.pallas import tpu_sc as plsc

N = 10000
E1 = 160000
EA = 320000
S2 = 5000
I2 = 50000
D = 128
L = 6
D0 = 57

NP = N + 16
K = 64
W = 128
AXF = 30720
EA_PAD = 327680
I1_PAD = 327680
I2_PAD = 65536
ADJ_CHUNKS = EA_PAD // (16 * K)
I1_CHUNKS = I1_PAD // (16 * K)
I2_CHUNKS = I2_PAD // (16 * K)
TROWS = 632
ACC_ROWS = 16 * TROWS


def _embed_body(x_ref, w_ref, b_ref, o_ref):
    o_ref[...] = jnp.dot(x_ref[...], w_ref[...],
                         preferred_element_type=jnp.float32,
                 precision=jax.lax.Precision.HIGHEST) + b_ref[...]


def _embed(x, w, b):
    return pl.pallas_call(
        _embed_body,
        grid=(10,),
        in_specs=[pl.BlockSpec((1000, D0), lambda i: (i, 0)),
                  pl.BlockSpec((D0, D), lambda i: (0, 0)),
                  pl.BlockSpec((D,), lambda i: (0,))],
        out_specs=pl.BlockSpec((1000, D), lambda i: (i, 0)),
        out_shape=jax.ShapeDtypeStruct((N, D), jnp.float32),
    )(x, w, b)


def _proj_body(h_ref, w_ref, b_ref, o_ref):
    o_ref[...] = jnp.dot(h_ref[...], w_ref[...],
                         preferred_element_type=jnp.float32,
                 precision=jax.lax.Precision.HIGHEST) + b_ref[...]


def _proj(h, wc, bc):
    return pl.pallas_call(
        _proj_body,
        grid=(10,),
        in_specs=[pl.BlockSpec((1000, D), lambda i: (i, 0)),
                  pl.BlockSpec((D, 6 * D), lambda i: (0, 0)),
                  pl.BlockSpec((6 * D,), lambda i: (0,))],
        out_specs=pl.BlockSpec((1000, 6 * D), lambda i: (i, 0)),
        out_shape=jax.ShapeDtypeStruct((N, 6 * D), jnp.float32),
    )(h, wc, bc)


def _q_body(e_ref, w_ref, o_ref):
    e = e_ref[...]
    w = w_ref[...]
    acc = e[:, 0:1] * w[0:1, :]
    for j in range(1, w.shape[0]):
        acc = acc + e[:, j:j + 1] * w[j:j + 1, :]
    o_ref[...] = acc


def _q(attr, wq, rows, block):
    nb = rows // block
    da = attr.shape[1]
    return pl.pallas_call(
        _q_body,
        grid=(nb,),
        in_specs=[pl.BlockSpec((block, da), lambda i: (i, 0)),
                  pl.BlockSpec((da, D), lambda i: (0, 0))],
        out_specs=pl.BlockSpec((block, D), lambda i: (i, 0)),
        out_shape=jax.ShapeDtypeStruct((rows, D), jnp.float32),
    )(attr, wq)


def _upd_body(h_ref, o1_ref, o2_ref, i1a_ref, i1b_ref, i2_ref, x_ref, ax_ref,
              wh_ref, bh_ref, ho_ref, xo_ref):
    h = h_ref[...]
    wh = wh_ref[...]
    hp = jax.lax.Precision.HIGHEST
    u = jnp.dot(h, wh[0:D], preferred_element_type=jnp.float32, precision=hp)
    u = u + jnp.dot(o1_ref[...], wh[D:2 * D],
                    preferred_element_type=jnp.float32, precision=hp)
    u = u + jnp.dot(o2_ref[...], wh[2 * D:3 * D],
                    preferred_element_type=jnp.float32, precision=hp)
    u = u + jnp.dot(i1a_ref[...] + i1b_ref[...], wh[3 * D:4 * D],
                    preferred_element_type=jnp.float32, precision=hp)
    u = u + jnp.dot(i2_ref[...], wh[4 * D:5 * D],
                    preferred_element_type=jnp.float32, precision=hp)
    u = jnp.maximum(u + bh_ref[...], 0.0)
    ho_ref[...] = h + u
    xo_ref[...] = x_ref[...] + 0.001 * ax_ref[...]


def _update(h, o1, o2, i1a, i1b, i2, x, axn, wh, bh):
    return pl.pallas_call(
        _upd_body,
        grid=(10,),
        in_specs=[pl.BlockSpec((1000, D), lambda i: (i, 0)),
                  pl.BlockSpec((1000, W), lambda i: (i, 0)),
                  pl.BlockSpec((1000, W), lambda i: (i, 0)),
                  pl.BlockSpec((1000, W), lambda i: (i, 0)),
                  pl.BlockSpec((1000, W), lambda i: (i, 0)),
                  pl.BlockSpec((1000, W), lambda i: (i, 0)),
                  pl.BlockSpec((1000, 3), lambda i: (i, 0)),
                  pl.BlockSpec((1000, 3), lambda i: (i, 0)),
                  pl.BlockSpec((5 * D, D), lambda i: (0, 0)),
                  pl.BlockSpec((D,), lambda i: (0,))],
        out_specs=[pl.BlockSpec((1000, D), lambda i: (i, 0)),
                   pl.BlockSpec((1000, 3), lambda i: (i, 0))],
        out_shape=[jax.ShapeDtypeStruct((N, D), jnp.float32),
                   jax.ShapeDtypeStruct((N, 3), jnp.float32)],
    )(h, o1, o2, i1a, i1b, i2, x, axn, wh, bh)


_SC_MESH = plsc.VectorSubcoreMesh(core_axis_name="c", subcore_axis_name="s",
                                  num_cores=2, num_subcores=16)
_SC_PARAMS = pltpu.CompilerParams(needs_layout_passes=False)

EPT = EA_PAD // 16
KG = 512
NGC = EPT // KG
I1A_CH = 128
I1B_CH = 192
I1B_BASE = 16 * I1A_CH * K


def _geom_body(s1, t1, s2, t2, Xp, g1, g2, Xv, sbig, tbig, d2g, dxg):
    cid = lax.axis_index("c")
    sid = lax.axis_index("s")
    pltpu.sync_copy(Xp, Xv)

    def pass_(s_hbm, t_hbm, g_hbm):
        base = sid * EPT
        pltpu.sync_copy(s_hbm.at[pl.ds(base, EPT)], sbig)
        pltpu.sync_copy(t_hbm.at[pl.ds(base, EPT)], tbig)

        def chunk(k, carry):
            off = k * KG
            for i in range(KG // 16):
                sv = sbig[pl.ds(off + i * 16, 16)]
                tv = tbig[pl.ds(off + i * 16, 16)]
                d2 = jnp.zeros((16,), jnp.float32)
                for c in range(3):
                    dx = (plsc.load_gather(Xv, [tv + c * NP])
                          - plsc.load_gather(Xv, [sv + c * NP]))
                    dxg[pl.ds(c * KG + i * 16, 16)] = dx
                    d2 = d2 + dx * dx
                d2g[pl.ds(i * 16, 16)] = d2
            pltpu.sync_copy(d2g, g_hbm.at[pl.ds(base + off, KG)])
            pltpu.sync_copy(
                dxg, g_hbm.at[pl.ds(EA_PAD + 3 * (base + off), 3 * KG)])
            return carry
        lax.fori_loop(0, NGC, chunk, 0)

    @pl.when(cid == 0)
    def _():
        pass_(s1, t1, g1)

    @pl.when(cid == 1)
    def _():
        pass_(s2, t2, g2)


_sc_geom = pl.kernel(
    _geom_body,
    out_type=[jax.ShapeDtypeStruct((4 * EA_PAD,), jnp.float32)
              for _ in range(2)],
    mesh=_SC_MESH,
    scratch_types=[
        pltpu.VMEM((3 * NP,), jnp.float32),
        pltpu.VMEM((EPT,), jnp.int32),
        pltpu.VMEM((EPT,), jnp.int32),
        pltpu.VMEM((KG,), jnp.float32),
        pltpu.VMEM((3 * KG,), jnp.float32),
    ],
    compiler_params=_SC_PARAMS,
)


def _msg_body(st1, t1_2d, st2, t2_2d, n1_2d, cn1, n2_2d, cn2,
              AB1, AB2, PQ1, PQ2, g1, g2, wpack,
              o1, o2, i1hA, i1hB, i2h, u1, u2,
              sbig, tbig, d2big, ubig, ab0, ab1, wv, acc,
              sga0, ssc0, sga1, ssc1):
    cid = lax.axis_index("c")
    sid = lax.axis_index("s")
    zero16 = jnp.zeros((16,), jnp.float32)
    lanes = lax.iota(jnp.int32, 16)
    lane0 = lanes == 0
    SC = 16
    SE = SC * K

    pltpu.sync_copy(wpack, wv)

    AB = (ab0, ab1)
    SGA = (sga0, sga1)
    SSC = (ssc0, ssc1)

    def zero_ar0():
        def row(e, carry):
            for j in range(8):
                ab0[e, pl.ds(j * 16, 16)] = zero16
            return carry
        lax.fori_loop(0, K, row, 0)

    def zero_acc():
        zsrc = ab0.at[pl.ds(0, K)]
        base = sid * TROWS
        for j in range(TROWS // K):
            pltpu.sync_copy(zsrc, acc.at[pl.ds(base + j * K, K)])
        rem = TROWS % K
        pltpu.sync_copy(ab0.at[pl.ds(0, rem)],
                        acc.at[pl.ds(base + TROWS - rem, rem)])

    def flush(out_ref):
        base = sid * TROWS
        pltpu.sync_copy(acc.at[pl.ds(base, TROWS)],
                        out_ref.at[pl.ds(base, TROWS)])

    def super_pass(nsuper, ebase, rbase, st_hbm, t2d_hbm, AB_hbm, compute,
                   g_hbm, u_hbm):
        dsrc = AB_hbm.at[pl.ds(0, 2 * K)]
        dsrc_k = AB_hbm.at[pl.ds(0, K)]

        def wait_g(p):
            pltpu.make_async_copy(dsrc, AB[p], SGA[p]).wait()

        def wait_s(p):
            pltpu.make_async_copy(dsrc_k, AB[p].at[pl.ds(K, K)],
                                  SSC[p]).wait()

        def fetch(cc, p):
            pltpu.async_copy(AB_hbm.at[sbig.at[pl.ds(cc * 2 * K, 2 * K)]],
                             AB[p], SGA[p])

        def super(j, carry):
            eb = ebase + j * SE
            pltpu.sync_copy(st_hbm.at[pl.ds(2 * eb, 2 * SE)], sbig)
            pltpu.sync_copy(
                t2d_hbm.at[pl.ds(pl.multiple_of(rbase + j * SC, 8), SC)],
                tbig)
            if g_hbm is not None:
                pltpu.sync_copy(g_hbm.at[pl.ds(eb, SE)], d2big)

            @pl.when(j > 0)
            def _():
                wait_s(0)
            fetch(0, 0)
            for cc in range(SC):
                p = cc % 2
                wait_g(p)
                if cc < SC - 1:
                    q = 1 - p
                    if cc == 0:
                        @pl.when(j > 0)
                        def _():
                            wait_s(1)
                    else:
                        wait_s(q)
                    fetch(cc + 1, q)
                compute(cc, AB[p])
                pltpu.async_copy(AB[p].at[pl.ds(K, K)], acc.at[tbig.at[cc]],
                                 SSC[p], add=True)
            if u_hbm is not None:
                for i in range(SE // 16):
                    uv = ubig[pl.ds(i * 16, 16)]
                    e2v = jnp.exp(-2.0 * jnp.abs(uv))
                    th = (1.0 - e2v) / (1.0 + e2v)
                    ubig[pl.ds(i * 16, 16)] = jnp.where(uv < 0.0, -th, th)
                pltpu.sync_copy(ubig, u_hbm.at[pl.ds(eb, SE)])
            return carry
        lax.fori_loop(0, nsuper, super, 0)
        wait_s(0)
        wait_s(1)

    def adj_pass(st_hbm, t2d_hbm, AB_hbm, g_hbm, u_hbm, wd2_row, wx_row):
        wd2 = [wv[wd2_row, pl.ds(j * 16, 16)] for j in range(8)]
        wx = [wv[wx_row, pl.ds(j * 16, 16)] for j in range(8)]

        def compute(cc, ab):
            def edge2(e2, ecarry):
                for half in range(2):
                    e = 2 * e2 + half
                    ev = jnp.full((16,), e + cc * K, jnp.int32)
                    d2v = plsc.load_gather(d2big, [ev])
                    u = zero16
                    for j in range(8):
                        m = jnp.maximum(
                            ab[e, pl.ds(j * 16, 16)]
                            + ab[K + e, pl.ds(j * 16, 16)]
                            + d2v * wd2[j], 0.0)
                        ab[K + e, pl.ds(j * 16, 16)] = m
                        u = u + m * wx[j]
                    plsc.store_scatter(ubig, [ev],
                                       jnp.full((16,), jnp.sum(u),
                                                jnp.float32),
                                       mask=lane0)
                return ecarry
            lax.fori_loop(0, K // 2, edge2, 0)

        super_pass(ADJ_CHUNKS // SC, sid * EPT, sid * (EPT // K),
                   st_hbm, t2d_hbm, AB_hbm, compute, g_hbm, u_hbm)

    def inc_pass(cn_hbm, n2d_hbm, PQ_hbm, nch, ebase):
        def compute(cc, ab):
            def edge(e, ecarry):
                for j in range(8):
                    ab[K + e, pl.ds(j * 16, 16)] = jnp.maximum(
                        ab[e, pl.ds(j * 16, 16)]
                        + ab[K + e, pl.ds(j * 16, 16)], 0.0)
                return ecarry
            lax.fori_loop(0, K, edge, 0)

        super_pass(nch // SC, ebase, ebase // K,
                   cn_hbm, n2d_hbm, PQ_hbm, compute, None, None)

    @pl.when(cid == 0)
    def _():
        zero_ar0()
        zero_acc()
        plsc.subcore_barrier()
        adj_pass(st1, t1_2d, AB1, g1, u1, 0, 1)
        plsc.subcore_barrier()
        flush(o1)
        plsc.subcore_barrier()
        zero_ar0()
        zero_acc()
        plsc.subcore_barrier()
        inc_pass(cn1, n1_2d, PQ1, I1A_CH, sid * (I1A_CH * K))
        plsc.subcore_barrier()
        flush(i1hA)
        plsc.subcore_barrier()
        zero_ar0()
        zero_acc()
        plsc.subcore_barrier()
        inc_pass(cn2, n2_2d, PQ2, I2_CHUNKS,
                 sid * (I2_CHUNKS * K))
        plsc.subcore_barrier()
        flush(i2h)

    @pl.when(cid == 1)
    def _():
        zero_ar0()
        zero_acc()
        plsc.subcore_barrier()
        adj_pass(st2, t2_2d, AB2, g2, u2, 2, 3)
        plsc.subcore_barrier()
        flush(o2)
        plsc.subcore_barrier()
        zero_ar0()
        zero_acc()
        plsc.subcore_barrier()
        inc_pass(cn1, n1_2d, PQ1, I1B_CH,
                 I1B_BASE + sid * (I1B_CH * K))
        plsc.subcore_barrier()
        flush(i1hB)


_sc_msg = pl.kernel(
    _msg_body,
    out_type=[jax.ShapeDtypeStruct((ACC_ROWS, W), jnp.float32)
              for _ in range(5)]
    + [jax.ShapeDtypeStruct((EA_PAD,), jnp.float32) for _ in range(2)],
    mesh=_SC_MESH,
    scratch_types=[
        pltpu.VMEM((2048,), jnp.int32),
        pltpu.VMEM((16, K), jnp.int32),
        pltpu.VMEM((1024,), jnp.float32),
        pltpu.VMEM((1024,), jnp.float32),
        pltpu.VMEM((2 * K, D), jnp.float32),
        pltpu.VMEM((2 * K, D), jnp.float32),
        pltpu.VMEM((4, D), jnp.float32),
        pltpu.VMEM_SHARED((ACC_ROWS, W), jnp.float32),
        pltpu.SemaphoreType.DMA,
        pltpu.SemaphoreType.DMA,
        pltpu.SemaphoreType.DMA,
        pltpu.SemaphoreType.DMA,
    ],
    compiler_params=_SC_PARAMS,
)


def _ax_body(t1, t2, u1, u2, g1, g2, ax1p, ax2p, axv, tbig, ubig, dxg):
    cid = lax.axis_index("c")
    sid = lax.axis_index("s")
    zero16 = jnp.zeros((16,), jnp.float32)
    lanes = lax.iota(jnp.int32, 16)
    axmask = lanes < 3
    plane = lanes * NP

    def pass_(t_hbm, u_hbm, g_hbm, axp):
        def zrow(i, carry):
            axv[pl.ds(i * 16, 16)] = zero16
            return carry
        lax.fori_loop(0, AXF // 16, zrow, 0)
        base = sid * EPT
        pltpu.sync_copy(t_hbm.at[pl.ds(base, EPT)], tbig)
        pltpu.sync_copy(u_hbm.at[pl.ds(base, EPT)], ubig)

        def chunk(k, carry):
            off = k * KG
            pltpu.sync_copy(
                g_hbm.at[pl.ds(EA_PAD + 3 * (base + off), 3 * KG)], dxg)

            def grp(i, ecarry):
                tv = tbig[pl.ds(off + i * 16, 16)]
                uv = ubig[pl.ds(off + i * 16, 16)]
                for c in range(3):
                    dv = dxg[pl.ds(c * KG + i * 16, 16)]
                    plsc.addupdate_scatter(axv, [tv + c * NP], uv * dv)
                return ecarry
            lax.fori_loop(0, KG // 16, grp, 0, unroll=2)
            return carry
        lax.fori_loop(0, NGC, chunk, 0)
        pltpu.sync_copy(axv, axp.at[sid])

    @pl.when(cid == 0)
    def _():
        pass_(t1, u1, g1, ax1p)

    @pl.when(cid == 1)
    def _():
        pass_(t2, u2, g2, ax2p)


_sc_ax = pl.kernel(
    _ax_body,
    out_type=[jax.ShapeDtypeStruct((16, AXF), jnp.float32)
              for _ in range(2)],
    mesh=_SC_MESH,
    scratch_types=[
        pltpu.VMEM((AXF,), jnp.float32),
        pltpu.VMEM((EPT,), jnp.int32),
        pltpu.VMEM((EPT,), jnp.float32),
        pltpu.VMEM((3 * KG,), jnp.float32),
    ],
    compiler_params=_SC_PARAMS,
)


def _pad_idx(a, n, fill):
    return jnp.concatenate(
        [a, jnp.full((n - a.shape[0],), fill, jnp.int32)])


def _pad_rows(a):
    return jnp.pad(a, ((0, 16), (0, 0)))


def kernel(x, pos, edge_attr, sse_attr, adj1_src, adj1_dst, adj2_src,
           adj2_dst, inc1_cell, inc1_node, inc2_cell, inc2_node,
           W0, b0, Wm1, bm1, Wx1, Wm2, bm2, Wx2, Wi1, bi1, Wi2, bi2, Wh, bh):
    H0 = _embed(x, W0, b0)
    X = pos

    s1 = _pad_idx(adj1_src, EA_PAD, N)
    t1 = _pad_idx(adj1_dst, EA_PAD, N)
    s2 = _pad_idx(adj2_src, EA_PAD, N)
    t2 = _pad_idx(adj2_dst, EA_PAD, N)
    c1 = _pad_idx(inc1_cell, I1_PAD, 0)
    n1 = _pad_idx(inc1_node, I1_PAD, N)
    c2 = _pad_idx(inc2_cell, I2_PAD, 0)
    n2 = _pad_idx(inc2_node, I2_PAD, N)
    t1_2d = t1.reshape(-1, K)
    t2_2d = t2.reshape(-1, K)
    n1_2d = n1.reshape(-1, K)
    n2_2d = n2.reshape(-1, K)
    def _comb(a, b, boff):
        return jnp.concatenate(
            [a.reshape(-1, K), b.reshape(-1, K) + boff], axis=1).reshape(-1)
    st1 = _comb(s1, t1, NP)
    st2 = _comb(s2, t2, NP)
    cn1 = _comb(c1, n1, E1)
    cn2 = _comb(c2, n2, S2)

    zD = jnp.zeros((D,), jnp.float32)
    for l in range(L):
        Wc = jnp.concatenate([
            Wm1[l, :D], Wm1[l, D:2 * D],
            Wm2[l, :D], Wm2[l, D:2 * D],
            Wi1[l, :D], Wi2[l, :D]], axis=1)
        bc = jnp.concatenate([zD, bm1[l], zD, bm2[l], bi1[l], bi2[l]])
        C = _proj(H0, Wc, bc)
        A1 = _pad_rows(C[:, 0:D])
        B1 = _pad_rows(C[:, D:2 * D])
        A2 = _pad_rows(C[:, 2 * D:3 * D])
        B2 = _pad_rows(C[:, 3 * D:4 * D])
        P1 = _pad_rows(C[:, 4 * D:5 * D])
        P2 = _pad_rows(C[:, 5 * D:6 * D])
        Q1 = _q(edge_attr, Wi1[l, D:], E1, 2000)
        Q2 = _q(sse_attr, Wi2[l, D:], S2, 5000)
        Xp = jnp.pad(X, ((0, 16), (0, 0))).T.reshape(-1)
        wpack = jnp.stack([Wm1[l, 2 * D], Wx1[l, :, 0],
                           Wm2[l, 2 * D], Wx2[l, :, 0]])

        g1, g2 = _sc_geom(s1, t1, s2, t2, Xp)
        AB1 = jnp.concatenate([A1, B1])
        AB2 = jnp.concatenate([A2, B2])
        PQ1 = jnp.concatenate([Q1, P1])
        PQ2 = jnp.concatenate([Q2, P2])
        o1, o2, i1hA, i1hB, i2h, u1, u2 = _sc_msg(
            st1, t1_2d, st2, t2_2d, n1_2d, cn1, n2_2d, cn2,
            AB1, AB2, PQ1, PQ2, g1, g2, wpack)
        ax1p, ax2p = _sc_ax(t1, t2, u1, u2, g1, g2)

        axf = ax1p.sum(axis=0) + ax2p.sum(axis=0)
        axn = axf[:3 * NP].reshape(3, NP)[:, :N].T

        H0, X = _update(H0, o1, o2, i1hA, i1hB, i2h, X, axn, Wh[l], bh[l])

    return H0, X
